# baseline (speedup 1.0000x reference)
import functools

import jax
import jax.numpy as jnp
from jax import lax
from jax.experimental import pallas as pl
from jax.experimental.pallas import tpu as pltpu
from jax.experimental.pallas import tpu_sc as plsc

_NW = 32
_B = 80
_L = 16
_R = 3
_RS = 2


def kernel(X):
    M, N = X.shape
    nblocks = M // _B
    iters = -(-nblocks // _NW)
    total = 6 * (-(-(iters + 2) // 6))
    mesh = plsc.VectorSubcoreMesh(core_axis_name="c", subcore_axis_name="s")

    @functools.partial(
        pl.kernel,
        mesh=mesh,
        out_type=[
            jax.ShapeDtypeStruct((M, 160), X.dtype),
            jax.ShapeDtypeStruct((M, 160), X.dtype),
        ],
        scratch_types=(
            [pltpu.VMEM((_B, 128), jnp.float32) for _ in range(_R)]
            + [pltpu.VMEM((_B, 128), jnp.float32) for _ in range(_R)]
            + [pltpu.VMEM((_B, 160), jnp.float32) for _ in range(_RS)]
            + [pltpu.VMEM((_B, 32), jnp.float32) for _ in range(_RS)]
            + [pltpu.SemaphoreType.DMA for _ in range(2 * _R + _RS)]
        ),
    )
    def run(x_hbm, p0_hbm, p1_hbm, *scratch):
        o = 0
        bufA = scratch[o:o + _R]; o += _R
        bufB = scratch[o:o + _R]; o += _R
        bufP1 = scratch[o:o + _RS]; o += _RS
        bufP0b = scratch[o:o + _RS]; o += _RS
        s_in = scratch[o:o + _R]; o += _R
        s_cf = scratch[o:o + _R]; o += _R
        s_out = scratch[o:o + _RS]; o += _RS
        wid = lax.axis_index("s") * 2 + lax.axis_index("c")

        def blk(i):
            return wid + i * _NW

        def rows_of(b):
            return pl.ds(b * _B, _B)

        def in_copies(b, r):
            rows = rows_of(b)
            return (
                pltpu.make_async_copy(x_hbm.at[rows, pl.ds(0, 128)], bufA[r], s_in[r]),
                pltpu.make_async_copy(x_hbm.at[rows, pl.ds(128, 128)], bufB[r], s_in[r]),
            )

        def cf_copy(b, r):
            return pltpu.make_async_copy(
                bufA[r], p0_hbm.at[rows_of(b), pl.ds(0, 128)], s_cf[r])

        def out_copies(b, r):
            rows = rows_of(b)
            return (
                pltpu.make_async_copy(bufP1[r], p1_hbm.at[rows], s_out[r]),
                pltpu.make_async_copy(bufP0b[r], p0_hbm.at[rows, pl.ds(128, 32)], s_out[r]),
            )

        def compute(r_in, r_st):
            srcA, srcB = bufA[r_in], bufB[r_in]
            dst1, dst0b = bufP1[r_st], bufP0b[r_st]

            @plsc.parallel_loop(0, _B, 1, unroll=8)
            def _rot(row):
                dst1[row, pl.ds(0, _L)] = srcA[row, pl.ds(96, _L)]
                dst1[row, pl.ds(_L, _L)] = srcA[row, pl.ds(112, _L)]
                for k in range(8):
                    v = srcB[row, pl.ds(k * _L, _L)]
                    dst1[row, pl.ds(32 + k * _L, _L)] = v
                    if k < 2:
                        dst0b[row, pl.ds(k * _L, _L)] = v

        def guarded(i, rk, f, mod=_R):
            b = blk(i)

            @pl.when(jnp.logical_and(i >= 0, b < nblocks))
            def _():
                f(b, rk % mod)

        def sub_iter(i, k):
            guarded(i, k, lambda b, r: [c.wait() for c in in_copies(b, r)])
            guarded(i, k, lambda b, r: cf_copy(b, r).start())
            guarded(i - _RS, k, lambda b, r: [c.wait() for c in out_copies(b, r)], mod=_RS)
            guarded(i, k, lambda b, r: compute(k % _R, r), mod=_RS)
            guarded(i, k, lambda b, r: [c.start() for c in out_copies(b, r)], mod=_RS)
            guarded(i - 1, k - 1, lambda b, r: cf_copy(b, r).wait())
            guarded(i + 2, k + 2, lambda b, r: [c.start() for c in in_copies(b, r)])

        for i in (0, 1):
            guarded(i, i, lambda b, r: [c.start() for c in in_copies(b, r)])

        def body(j, carry):
            for k in range(6):
                sub_iter(6 * j + k, k)
            return carry

        lax.fori_loop(0, total // 6, body, 0)

        for i in range(total - _RS, total):
            guarded(i, i, lambda b, r: [c.wait() for c in out_copies(b, r)], mod=_RS)
        guarded(total - 1, total - 1, lambda b, r: cf_copy(b, r).wait())

    p0, p1 = run(X)
    return (p0, p1)
